# speedup vs baseline: 2443.1737x; 2443.1737x over previous
import jax
import jax.numpy as jnp
from jax import lax
from jax.experimental import pallas as pl
from jax.experimental.pallas import tpu as pltpu

_B, _N, _DIN, _H, _DOUT = 4, 128, 512, 256, 128
_PAIR_TILE = 8


def _sigmoid(x):
    return 1.0 / (1.0 + jnp.exp(-x))


def _outer(a_row, b_row):
    return lax.dot_general(a_row, b_row, (((0,), (0,)), ((), ())),
                           preferred_element_type=jnp.float32)


def _graph_kernel(x_ref, dep_ref, Win_ref, bin_ref, Wg1_ref, bg1_ref,
                  Wg2_ref, bg2_ref, Wg3_ref, bg3_ref, Wo1_ref, bo1_ref,
                  Wo2_ref, bo2_ref, We1_ref, be1_ref, We2t_ref, be2_ref,
                  ne_ref, p_ref, adj_ref,
                  R_ref, Rt_ref, C_ref, adj2_ref):
    f32 = jnp.float32
    x = x_ref[0]
    dep = dep_ref[0]

    row = lax.broadcasted_iota(jnp.int32, (_N, _N), 0)
    col = lax.broadcasted_iota(jnp.int32, (_N, _N), 1)
    eye = (row == col).astype(f32)

    dep_p = _sigmoid(dep)
    Afwd = jnp.where((dep_p > 0.5) & (row > col), 1.0, 0.0).astype(f32)
    chain = (row == col + 1).astype(f32)
    A = jnp.where(jnp.sum(Afwd) > 0.0, Afwd, chain)
    At = A + eye
    dis = 1.0 / jnp.sqrt(jnp.sum(At, axis=1))
    normA = At * dis[:, None] * dis[None, :]

    def mm(a, b):
        return jnp.dot(a, b, preferred_element_type=f32)

    feats = mm(x, Win_ref[...]) + bin_ref[...]
    h = jnp.maximum(mm(normA, mm(feats, Wg1_ref[...])) + bg1_ref[...], 0.0)
    h = jnp.maximum(mm(normA, mm(h, Wg2_ref[...])) + bg2_ref[...], 0.0)
    h = mm(normA, mm(h, Wg3_ref[...])) + bg3_ref[...]
    ne = mm(jnp.maximum(mm(h, Wo1_ref[...]) + bo1_ref[...], 0.0),
            Wo2_ref[...]) + bo2_ref[...]
    ne_ref[0] = ne

    U = mm(ne, We1_ref[: _DOUT, :]) + be1_ref[...]
    V = mm(ne, We1_ref[_DOUT:, :])
    we2 = We2t_ref[...]
    be2 = be2_ref[0, 0]
    tiles = []
    for t in range(_N // _PAIR_TILE):
        u = U[t * _PAIR_TILE:(t + 1) * _PAIR_TILE]
        m = jnp.maximum(u[:, None, :] + V[None, :, :], 0.0)
        tiles.append(jnp.sum(m * we2[None, :, :], axis=-1) + be2)
    logits = jnp.concatenate(tiles, axis=0)
    p = _sigmoid(logits) * (1.0 - eye)
    p_ref[0] = p

    adj0 = lax.dot_general(Afwd, eye, (((0,), (0,)), ((), ())),
                           preferred_element_type=f32)
    adj2_ref[...] = adj0
    Cm = ((p > 0.7) & (adj0 == 0.0) & (row != col)).astype(f32)
    ncand = jnp.sum(Cm)

    @pl.when(ncand > 0.0)
    def _greedy():
        def closure(M):
            for _ in range(7):
                M = jnp.where(M + mm(M, M) > 0.5, 1.0, 0.0)
            return M

        R_ref[...] = closure(adj0)
        Rt_ref[...] = closure(Afwd)
        C_ref[...] = Cm
        col1 = lax.broadcasted_iota(jnp.int32, (1, _N), 1)

        def row_body(i, carry):
            c_row = C_ref[pl.ds(i, 1), :]
            rt_row = Rt_ref[pl.ds(i, 1), :]
            accept = c_row * (1.0 - rt_row)
            na = jnp.sum(accept)

            @pl.when(na > 0.0)
            def _apply():
                adj2_ref[pl.ds(i, 1), :] = adj2_ref[pl.ds(i, 1), :] + accept
                d = mm(accept, R_ref[...])
                desc = jnp.where(d + accept > 0.5, 1.0, 0.0)
                anc = jnp.maximum(rt_row, (col1 == i).astype(f32))
                R_ref[...] = jnp.maximum(R_ref[...], _outer(anc, desc))
                Rt_ref[...] = jnp.maximum(Rt_ref[...], _outer(desc, anc))

            return carry

        lax.fori_loop(0, _N, row_body, 0)

    adj_ref[0] = adj2_ref[...]


@jax.jit
def kernel(subtask_embeddings, dependencies, W_in, b_in, W_g1, b_g1,
           W_g2, b_g2, W_g3, b_g3, W_o1, b_o1, W_o2, b_o2,
           W_e1, b_e1, W_e2, b_e2):
    f32 = jnp.float32
    b2 = lambda v: v.reshape(1, -1).astype(f32)

    bspec = lambda shp: pl.BlockSpec(shp, lambda b: (b, 0, 0))
    wspec = lambda shp: pl.BlockSpec(shp, lambda b, _n=None: tuple(0 for _ in shp))

    out_shapes = (
        jax.ShapeDtypeStruct((_B, _N, _DOUT), f32),
        jax.ShapeDtypeStruct((_B, _N, _N), f32),
        jax.ShapeDtypeStruct((_B, _N, _N), f32),
    )
    ne, p, adj = pl.pallas_call(
        _graph_kernel,
        grid=(_B,),
        in_specs=[
            bspec((1, _N, _DIN)),
            bspec((1, _N, _N)),
            wspec((_DIN, _H)), wspec((1, _H)),
            wspec((_H, _H)), wspec((1, _H)),
            wspec((_H, _H)), wspec((1, _H)),
            wspec((_H, _H)), wspec((1, _H)),
            wspec((_H, _H)), wspec((1, _H)),
            wspec((_H, _DOUT)), wspec((1, _DOUT)),
            wspec((2 * _DOUT, _H)), wspec((1, _H)),
            wspec((1, _H)), wspec((1, 1)),
        ],
        out_specs=[bspec((1, _N, _DOUT)), bspec((1, _N, _N)),
                   bspec((1, _N, _N))],
        out_shape=out_shapes,
        scratch_shapes=[
            pltpu.VMEM((_N, _N), f32),
            pltpu.VMEM((_N, _N), f32),
            pltpu.VMEM((_N, _N), f32),
            pltpu.VMEM((_N, _N), f32),
        ],
    )(subtask_embeddings, dependencies,
      W_in, b2(b_in), W_g1, b2(b_g1), W_g2, b2(b_g2), W_g3, b2(b_g3),
      W_o1, b2(b_o1), W_o2, b2(b_o2), W_e1, b2(b_e1),
      W_e2.reshape(1, _H), b_e2.reshape(1, 1))
    return ne, p, adj
